# baseline (speedup 1.0000x reference)
import functools
import math

import jax
import jax.numpy as jnp
from jax import lax
from jax.experimental import pallas as pl
from jax.experimental.pallas import tpu as pltpu
from jax.experimental.pallas import tpu_sc as plsc

_N = 4096
_V = 32000
_PAD = 0
_SMOOTH = 0.1
_CONF = 1.0 - _SMOOTH
_EPS = _SMOOTH / (_V - 2)
_C = _CONF * math.log(_CONF) + _SMOOTH * math.log(_EPS)
_COEF_T = _CONF - _EPS

_LANES = 16
_NUM_WORKERS = 32

_NSC = 1280
_NTC = _N - _NSC

_ROWS_PER_STEP = 128
_NUM_STEPS = _NTC // _ROWS_PER_STEP

_ROWS_PER_WORKER = _NSC // _NUM_WORKERS
_GROUPS = _ROWS_PER_WORKER // 8
_CHUNKS_PER_WORKER = 2 * _GROUPS
_CHUNK = 4 * _V
_HALF = _V // 2
_TILES = _CHUNK // 1024


def _sc_chunk_stats(buf, statm_v, stats_v, statp_v):
    for sp in range(4):
        s0, s1 = 2 * sp, 2 * sp + 1

        def pass1(k, carry, s0=s0, s1=s1):
            m0, p0, m1, p1 = carry
            for u in range(8):
                x = buf[pl.ds(k * 1024 + s0 * 128 + u * _LANES, _LANES)]
                y = buf[pl.ds(k * 1024 + s1 * 128 + u * _LANES, _LANES)]
                m0 = jnp.maximum(m0, x)
                p0 = p0 + x
                m1 = jnp.maximum(m1, y)
                p1 = p1 + y
            return m0, p0, m1, p1

        zero = jnp.zeros((_LANES,), jnp.float32)
        ninf = jnp.full((_LANES,), -jnp.inf, jnp.float32)
        m0, p0, m1, p1 = lax.fori_loop(0, _TILES, pass1,
                                       (ninf, zero, ninf, zero))

        def pass2(k, carry, s0=s0, s1=s1, m0=m0, m1=m1):
            a0, a1 = carry
            for u in range(8):
                x = buf[pl.ds(k * 1024 + s0 * 128 + u * _LANES, _LANES)]
                y = buf[pl.ds(k * 1024 + s1 * 128 + u * _LANES, _LANES)]
                a0 = a0 + jnp.exp(x - m0)
                a1 = a1 + jnp.exp(y - m1)
            return a0, a1

        a0, a1 = lax.fori_loop(0, _TILES, pass2, (zero, zero))
        statm_v[pl.ds(s0 * _LANES, _LANES)] = m0
        stats_v[pl.ds(s0 * _LANES, _LANES)] = a0
        statp_v[pl.ds(s0 * _LANES, _LANES)] = p0
        statm_v[pl.ds(s1 * _LANES, _LANES)] = m1
        stats_v[pl.ds(s1 * _LANES, _LANES)] = a1
        statp_v[pl.ds(s1 * _LANES, _LANES)] = p1


def _sc_dense_body(pred_hbm, tgt_hbm, outm_hbm, outs_hbm, outp_hbm, outg_hbm,
                   buf, tgt_v, statm_v, stats_v, statp_v, accg_v, sem, semt):
    wid = lax.axis_index("s") * 2 + lax.axis_index("c")
    row0 = _NTC + wid * _ROWS_PER_WORKER
    for gi in range(_GROUPS):
        pltpu.sync_copy(tgt_hbm.at[pl.ds(row0 + gi * 8, 8)],
                        tgt_v.at[pl.ds(gi * _LANES, 8)])
    lane = lax.iota(jnp.int32, _LANES)

    def group(gi, accg):
        tvec = tgt_v[pl.ds(gi * _LANES, _LANES)]
        for h in range(2):
            r0 = row0 + 8 * gi + 4 * h
            cps = [pltpu.async_copy(pred_hbm.at[r0 + rr],
                                    buf.at[pl.ds(rr * _V, _V)], sem)
                   for rr in range(4)]
            for cp in cps:
                cp.wait()
            for s in range(8):
                t = tvec[s]
                ct = t - _HALF * h
                in_chunk = (ct >= 0) & (ct < _HALF) & (t != _PAD)
                ct = jnp.where(in_chunk, ct, 0)
                k = ct // 128
                addr = k * 1024 + s * 128 + (ct - k * 128)
                v = buf[pl.ds(addr, _LANES)]
                gt = v[0]
                accg = accg + jnp.where(in_chunk, -_COEF_T * gt, 0.0)
                if h == 0:
                    v0 = buf[pl.ds(s * 128, _LANES)]
                    accg = accg + jnp.where(t != _PAD, _EPS * v0[0], 0.0)
            _sc_chunk_stats(buf, statm_v, stats_v, statp_v)
            base = (wid * _CHUNKS_PER_WORKER + 2 * gi + h) * 128
            pltpu.sync_copy(statm_v, outm_hbm.at[pl.ds(base, 128)])
            pltpu.sync_copy(stats_v, outs_hbm.at[pl.ds(base, 128)])
            pltpu.sync_copy(statp_v, outp_hbm.at[pl.ds(base, 128)])
        return accg

    accg = lax.fori_loop(0, _GROUPS, group, jnp.float32(0.0))
    statm_v[pl.ds(0, _LANES)] = jnp.where(lane == 0, accg, 0.0)
    pltpu.sync_copy(statm_v.at[pl.ds(0, _LANES)],
                    outg_hbm.at[pl.ds(wid * _LANES, _LANES)])


@functools.lru_cache(maxsize=None)
def _sc_dense_fn():
    stats = jax.ShapeDtypeStruct((_NSC * 2 * _LANES,), jnp.float32)
    return pl.kernel(
        _sc_dense_body,
        mesh=plsc.VectorSubcoreMesh(core_axis_name="c", subcore_axis_name="s"),
        out_type=(stats, stats, stats,
                  jax.ShapeDtypeStruct((_NUM_WORKERS * _LANES,),
                                       jnp.float32)),
        scratch_types=[
            pltpu.VMEM((_CHUNK + _LANES,), jnp.float32),
            pltpu.VMEM((_GROUPS * _LANES,), jnp.int32),
            pltpu.VMEM((128,), jnp.float32),
            pltpu.VMEM((128,), jnp.float32),
            pltpu.VMEM((128,), jnp.float32),
            pltpu.VMEM((_LANES,), jnp.float32),
            pltpu.SemaphoreType.DMA,
            pltpu.SemaphoreType.DMA,
        ],
    )


def _tc_main_body(tgt_ref, x_ref, out_ref):
    i = pl.program_id(0)
    x = x_ref[...]
    m = jnp.max(x, axis=1, keepdims=True)
    se = jnp.sum(jnp.exp(x - m), axis=1, keepdims=True)
    lse = m + jnp.log(se)
    p = jnp.sum(x, axis=1, keepdims=True)
    tgt = tgt_ref[...]
    cols = lax.broadcasted_iota(jnp.int32, (_ROWS_PER_STEP, _V), 1)
    w = jnp.where(cols == tgt, -_COEF_T, 0.0) + jnp.where(cols == 0, _EPS, 0.0)
    g = jnp.sum(w * x, axis=1, keepdims=True)
    mask = (tgt != _PAD).astype(jnp.float32)
    part = jnp.sum(mask * (_C + lse - _EPS * p + g))
    prev = jnp.where(i == 0, 0.0, out_ref[0, 0])
    out_ref[0, 0] = prev + part


def _tc_main(pred, tgt2d):
    return pl.pallas_call(
        _tc_main_body,
        grid=(_NUM_STEPS,),
        in_specs=[
            pl.BlockSpec((_ROWS_PER_STEP, 1), lambda i: (i, 0)),
            pl.BlockSpec((_ROWS_PER_STEP, _V), lambda i: (i, 0)),
        ],
        out_specs=pl.BlockSpec(memory_space=pltpu.SMEM),
        out_shape=jax.ShapeDtypeStruct((1, 1), jnp.float32),
        compiler_params=pltpu.CompilerParams(
            dimension_semantics=("arbitrary",)),
    )(tgt2d, pred)


def _tc_combine_body(tc_ref, g_ref, m_ref, s_ref, p_ref, tgt_ref, out_ref):
    m2 = m_ref[...]
    mrow = jnp.max(m2, axis=1, keepdims=True)
    se = jnp.sum(s_ref[...] * jnp.exp(m2 - mrow), axis=1, keepdims=True)
    lse = mrow + jnp.log(se)
    p = jnp.sum(p_ref[...], axis=1, keepdims=True)
    mask = (tgt_ref[...] != _PAD).astype(jnp.float32)
    part = jnp.sum(mask * (_C + lse - _EPS * p))
    tot = tc_ref[0, 0] + jnp.sum(g_ref[...]) + part
    out_ref[0, 0] = tot * (1.0 / _N)


def _tc_combine(tc_scalar, sc_g, m2, s2, p2, tgt_sc):
    return pl.pallas_call(
        _tc_combine_body,
        in_specs=[
            pl.BlockSpec(memory_space=pltpu.SMEM),
            pl.BlockSpec((_NUM_WORKERS, _LANES), lambda: (0, 0)),
            pl.BlockSpec((_NSC, 2 * _LANES), lambda: (0, 0)),
            pl.BlockSpec((_NSC, 2 * _LANES), lambda: (0, 0)),
            pl.BlockSpec((_NSC, 2 * _LANES), lambda: (0, 0)),
            pl.BlockSpec((_NSC, 1), lambda: (0, 0)),
        ],
        out_specs=pl.BlockSpec(memory_space=pltpu.SMEM),
        out_shape=jax.ShapeDtypeStruct((1, 1), jnp.float32),
    )(tc_scalar, sc_g, m2, s2, p2, tgt_sc)


def _stats_2d(a):
    return a.reshape(_NUM_WORKERS, _GROUPS, 2, 8, _LANES).transpose(
        0, 1, 3, 2, 4).reshape(_NSC, 2 * _LANES)


def kernel(pred, target):
    m2, s2, p2, sc_g = _sc_dense_fn()(pred, target)
    tc_scalar = _tc_main(pred, target.reshape(_N, 1))
    loss = _tc_combine(
        tc_scalar, sc_g.reshape(_NUM_WORKERS, _LANES),
        _stats_2d(m2), _stats_2d(s2), _stats_2d(p2),
        target[_NTC:].reshape(_NSC, 1))
    return loss[0, 0]


# speedup vs baseline: 1.1088x; 1.1088x over previous
import functools
import math

import jax
import jax.numpy as jnp
from jax import lax
from jax.experimental import pallas as pl
from jax.experimental.pallas import tpu as pltpu
from jax.experimental.pallas import tpu_sc as plsc

_N = 4096
_V = 32000
_PAD = 0
_SMOOTH = 0.1
_CONF = 1.0 - _SMOOTH
_EPS = _SMOOTH / (_V - 2)
_C = _CONF * math.log(_CONF) + _SMOOTH * math.log(_EPS)
_COEF_T = _CONF - _EPS

_LANES = 16
_NUM_WORKERS = 32

_NSC = 1024
_NTC = _N - _NSC

_ROWS_PER_STEP = 128
_NUM_STEPS = _NTC // _ROWS_PER_STEP

_ROWS_PER_WORKER = _NSC // _NUM_WORKERS
_GROUPS = _ROWS_PER_WORKER // 8
_CHUNKS_PER_WORKER = 2 * _GROUPS
_CHUNK = 4 * _V
_HALF = _V // 2
_TILES = _CHUNK // 1024


def _sc_chunk_stats(buf, statm_v, stats_v, statp_v, h):
    for sp in range(4):
        s0, s1 = 2 * sp, 2 * sp + 1

        def pass1(k, carry, s0=s0, s1=s1):
            m0, p0, m1, p1 = carry
            for u in range(8):
                x = buf[pl.ds(k * 1024 + s0 * 128 + u * _LANES, _LANES)]
                y = buf[pl.ds(k * 1024 + s1 * 128 + u * _LANES, _LANES)]
                m0 = jnp.maximum(m0, x)
                p0 = p0 + x
                m1 = jnp.maximum(m1, y)
                p1 = p1 + y
            return m0, p0, m1, p1

        zero = jnp.zeros((_LANES,), jnp.float32)
        ninf = jnp.full((_LANES,), -jnp.inf, jnp.float32)
        m0, p0, m1, p1 = lax.fori_loop(0, _TILES, pass1,
                                       (ninf, zero, ninf, zero))

        def pass2(k, carry, s0=s0, s1=s1, m0=m0, m1=m1):
            a0, a1 = carry
            for u in range(8):
                x = buf[pl.ds(k * 1024 + s0 * 128 + u * _LANES, _LANES)]
                y = buf[pl.ds(k * 1024 + s1 * 128 + u * _LANES, _LANES)]
                a0 = a0 + jnp.exp(x - m0)
                a1 = a1 + jnp.exp(y - m1)
            return a0, a1

        a0, a1 = lax.fori_loop(0, _TILES, pass2, (zero, zero))
        statm_v[pl.ds(s0 * 32 + h * _LANES, _LANES)] = m0
        stats_v[pl.ds(s0 * 32 + h * _LANES, _LANES)] = a0
        statp_v[pl.ds(s0 * 32 + h * _LANES, _LANES)] = p0
        statm_v[pl.ds(s1 * 32 + h * _LANES, _LANES)] = m1
        stats_v[pl.ds(s1 * 32 + h * _LANES, _LANES)] = a1
        statp_v[pl.ds(s1 * 32 + h * _LANES, _LANES)] = p1


def _sc_dense_body(pred_hbm, tgt_hbm, outm_hbm, outs_hbm, outp_hbm, outg_hbm,
                   buf, tgt_v, statm_v, stats_v, statp_v, accg_v, sem, semt):
    wid = lax.axis_index("s") * 2 + lax.axis_index("c")
    row0 = _NTC + wid * _ROWS_PER_WORKER
    for gi in range(_GROUPS):
        pltpu.sync_copy(tgt_hbm.at[pl.ds(row0 + gi * 8, 8)],
                        tgt_v.at[pl.ds(gi * _LANES, 8)])
    lane = lax.iota(jnp.int32, _LANES)

    def group(gi, accg):
        tvec = tgt_v[pl.ds(gi * _LANES, _LANES)]
        for h in range(2):
            r0 = row0 + 8 * gi + 4 * h
            cps = [pltpu.async_copy(pred_hbm.at[r0 + rr],
                                    buf.at[pl.ds(rr * _V, _V)], sem)
                   for rr in range(4)]
            for cp in cps:
                cp.wait()
            for s in range(8):
                t = tvec[s]
                ct = t - _HALF * h
                in_chunk = (ct >= 0) & (ct < _HALF) & (t != _PAD)
                ct = jnp.where(in_chunk, ct, 0)
                k = ct // 128
                addr = k * 1024 + s * 128 + (ct - k * 128)
                v = buf[pl.ds(addr, _LANES)]
                gt = v[0]
                accg = accg + jnp.where(in_chunk, -_COEF_T * gt, 0.0)
                if h == 0:
                    v0 = buf[pl.ds(s * 128, _LANES)]
                    accg = accg + jnp.where(t != _PAD, _EPS * v0[0], 0.0)
            _sc_chunk_stats(buf, statm_v, stats_v, statp_v, h)
        base = (wid * _GROUPS + gi) * 256
        pltpu.sync_copy(statm_v, outm_hbm.at[pl.ds(base, 256)])
        pltpu.sync_copy(stats_v, outs_hbm.at[pl.ds(base, 256)])
        pltpu.sync_copy(statp_v, outp_hbm.at[pl.ds(base, 256)])
        return accg

    accg = lax.fori_loop(0, _GROUPS, group, jnp.float32(0.0))
    statm_v[pl.ds(0, _LANES)] = jnp.where(lane == 0, accg, 0.0)
    pltpu.sync_copy(statm_v.at[pl.ds(0, _LANES)],
                    outg_hbm.at[pl.ds(wid * _LANES, _LANES)])


@functools.lru_cache(maxsize=None)
def _sc_dense_fn():
    stats = jax.ShapeDtypeStruct((_NSC * 2 * _LANES,), jnp.float32)
    return pl.kernel(
        _sc_dense_body,
        mesh=plsc.VectorSubcoreMesh(core_axis_name="c", subcore_axis_name="s"),
        out_type=(stats, stats, stats,
                  jax.ShapeDtypeStruct((_NUM_WORKERS * _LANES,),
                                       jnp.float32)),
        scratch_types=[
            pltpu.VMEM((_CHUNK + _LANES,), jnp.float32),
            pltpu.VMEM((_GROUPS * _LANES,), jnp.int32),
            pltpu.VMEM((256,), jnp.float32),
            pltpu.VMEM((256,), jnp.float32),
            pltpu.VMEM((256,), jnp.float32),
            pltpu.VMEM((_LANES,), jnp.float32),
            pltpu.SemaphoreType.DMA,
            pltpu.SemaphoreType.DMA,
        ],
    )


def _tc_main_body(tgt_ref, x_ref, out_ref):
    i = pl.program_id(0)
    x = x_ref[...]
    m = jnp.max(x, axis=1, keepdims=True)
    se = jnp.sum(jnp.exp(x - m), axis=1, keepdims=True)
    lse = m + jnp.log(se)
    p = jnp.sum(x, axis=1, keepdims=True)
    tgt = tgt_ref[...]
    cols = lax.broadcasted_iota(jnp.int32, (_ROWS_PER_STEP, _V), 1)
    w = jnp.where(cols == tgt, -_COEF_T, 0.0) + jnp.where(cols == 0, _EPS, 0.0)
    g = jnp.sum(w * x, axis=1, keepdims=True)
    mask = (tgt != _PAD).astype(jnp.float32)
    part = jnp.sum(mask * (_C + lse - _EPS * p + g))
    prev = jnp.where(i == 0, 0.0, out_ref[0, 0])
    out_ref[0, 0] = prev + part


def _tc_main(pred, tgt2d):
    return pl.pallas_call(
        _tc_main_body,
        grid=(_NUM_STEPS,),
        in_specs=[
            pl.BlockSpec((_ROWS_PER_STEP, 1), lambda i: (i, 0)),
            pl.BlockSpec((_ROWS_PER_STEP, _V), lambda i: (i, 0)),
        ],
        out_specs=pl.BlockSpec(memory_space=pltpu.SMEM),
        out_shape=jax.ShapeDtypeStruct((1, 1), jnp.float32),
        compiler_params=pltpu.CompilerParams(
            dimension_semantics=("arbitrary",)),
    )(tgt2d, pred)


def _tc_combine_body(tc_ref, g_ref, m_ref, s_ref, p_ref, tgt_ref, out_ref):
    m2 = m_ref[...]
    mrow = jnp.max(m2, axis=1, keepdims=True)
    se = jnp.sum(s_ref[...] * jnp.exp(m2 - mrow), axis=1, keepdims=True)
    lse = mrow + jnp.log(se)
    p = jnp.sum(p_ref[...], axis=1, keepdims=True)
    mask = (tgt_ref[...] != _PAD).astype(jnp.float32)
    part = jnp.sum(mask * (_C + lse - _EPS * p))
    tot = tc_ref[0, 0] + jnp.sum(g_ref[...]) + part
    out_ref[0, 0] = tot * (1.0 / _N)


def _tc_combine(tc_scalar, sc_g, m2, s2, p2, tgt_sc):
    return pl.pallas_call(
        _tc_combine_body,
        grid=(1,),
        in_specs=[
            pl.BlockSpec(memory_space=pltpu.SMEM),
            pl.BlockSpec((_NUM_WORKERS, _LANES), lambda i: (0, 0)),
            pl.BlockSpec((_NSC, 2 * _LANES), lambda i: (0, 0)),
            pl.BlockSpec((_NSC, 2 * _LANES), lambda i: (0, 0)),
            pl.BlockSpec((_NSC, 2 * _LANES), lambda i: (0, 0)),
            pl.BlockSpec((_NSC, 1), lambda i: (_NTC // _NSC, 0)),
        ],
        out_specs=pl.BlockSpec(memory_space=pltpu.SMEM),
        out_shape=jax.ShapeDtypeStruct((1, 1), jnp.float32),
    )(tc_scalar, sc_g, m2, s2, p2, tgt_sc)


def _stats_2d(a):
    return a.reshape(_NSC, 2 * _LANES)


def kernel(pred, target):
    tgt2d = target.reshape(_N, 1)
    m2, s2, p2, sc_g = _sc_dense_fn()(pred, target)
    tc_scalar = _tc_main(pred, tgt2d)
    loss = _tc_combine(
        tc_scalar, sc_g.reshape(_NUM_WORKERS, _LANES),
        _stats_2d(m2), _stats_2d(s2), _stats_2d(p2), tgt2d)
    return loss[0, 0]


# speedup vs baseline: 1.1108x; 1.0018x over previous
import functools
import math

import jax
import jax.numpy as jnp
from jax import lax
from jax.experimental import pallas as pl
from jax.experimental.pallas import tpu as pltpu
from jax.experimental.pallas import tpu_sc as plsc

_N = 4096
_V = 32000
_PAD = 0
_SMOOTH = 0.1
_CONF = 1.0 - _SMOOTH
_EPS = _SMOOTH / (_V - 2)
_C = _CONF * math.log(_CONF) + _SMOOTH * math.log(_EPS)
_COEF_T = _CONF - _EPS

_LANES = 16
_NUM_WORKERS = 32

_NSC = 1024
_NTC = _N - _NSC

_ROWS_PER_STEP = 128
_NUM_STEPS = _NTC // _ROWS_PER_STEP

_ROWS_PER_WORKER = _NSC // _NUM_WORKERS
_GROUPS = _ROWS_PER_WORKER // 8
_CHUNKS_PER_WORKER = 2 * _GROUPS
_CHUNK = 4 * _V
_HALF = _V // 2
_TILES = _CHUNK // 1024


def _sc_chunk_stats(buf, statm_v, stats_v, statp_v, h):
    for sp in range(4):
        s0, s1 = 2 * sp, 2 * sp + 1

        def pass1(k, carry, s0=s0, s1=s1):
            m0, p0, m1, p1 = carry
            xs, ys = [], []
            for u in range(8):
                x = buf[pl.ds(k * 1024 + s0 * 128 + u * _LANES, _LANES)]
                y = buf[pl.ds(k * 1024 + s1 * 128 + u * _LANES, _LANES)]
                m0 = jnp.maximum(m0, x)
                m1 = jnp.maximum(m1, y)
                xs.append(x)
                ys.append(y)
            while len(xs) > 1:
                xs = [a + b for a, b in zip(xs[::2], xs[1::2])]
                ys = [a + b for a, b in zip(ys[::2], ys[1::2])]
            return m0, p0 + xs[0], m1, p1 + ys[0]

        zero = jnp.zeros((_LANES,), jnp.float32)
        ninf = jnp.full((_LANES,), -jnp.inf, jnp.float32)
        m0, p0, m1, p1 = lax.fori_loop(0, _TILES, pass1,
                                       (ninf, zero, ninf, zero))

        def pass2(k, carry, s0=s0, s1=s1, m0=m0, m1=m1):
            a0, a1 = carry
            xs, ys = [], []
            for u in range(8):
                x = buf[pl.ds(k * 1024 + s0 * 128 + u * _LANES, _LANES)]
                y = buf[pl.ds(k * 1024 + s1 * 128 + u * _LANES, _LANES)]
                xs.append(jnp.exp(x - m0))
                ys.append(jnp.exp(y - m1))
            while len(xs) > 1:
                xs = [a + b for a, b in zip(xs[::2], xs[1::2])]
                ys = [a + b for a, b in zip(ys[::2], ys[1::2])]
            return a0 + xs[0], a1 + ys[0]

        a0, a1 = lax.fori_loop(0, _TILES, pass2, (zero, zero))
        statm_v[pl.ds(s0 * 32 + h * _LANES, _LANES)] = m0
        stats_v[pl.ds(s0 * 32 + h * _LANES, _LANES)] = a0
        statp_v[pl.ds(s0 * 32 + h * _LANES, _LANES)] = p0
        statm_v[pl.ds(s1 * 32 + h * _LANES, _LANES)] = m1
        stats_v[pl.ds(s1 * 32 + h * _LANES, _LANES)] = a1
        statp_v[pl.ds(s1 * 32 + h * _LANES, _LANES)] = p1


def _sc_dense_body(pred_hbm, tgt_hbm, outm_hbm, outs_hbm, outp_hbm, outg_hbm,
                   buf, tgt_v, statm_v, stats_v, statp_v, accg_v, sem, semt):
    wid = lax.axis_index("s") * 2 + lax.axis_index("c")
    row0 = _NTC + wid * _ROWS_PER_WORKER
    for gi in range(_GROUPS):
        pltpu.sync_copy(tgt_hbm.at[pl.ds(row0 + gi * 8, 8)],
                        tgt_v.at[pl.ds(gi * _LANES, 8)])
    lane = lax.iota(jnp.int32, _LANES)

    def group(gi, accg):
        tvec = tgt_v[pl.ds(gi * _LANES, _LANES)]
        for h in range(2):
            r0 = row0 + 8 * gi + 4 * h
            cps = [pltpu.async_copy(pred_hbm.at[r0 + rr],
                                    buf.at[pl.ds(rr * _V, _V)], sem)
                   for rr in range(4)]
            for cp in cps:
                cp.wait()
            for s in range(8):
                t = tvec[s]
                ct = t - _HALF * h
                in_chunk = (ct >= 0) & (ct < _HALF) & (t != _PAD)
                ct = jnp.where(in_chunk, ct, 0)
                k = ct // 128
                addr = k * 1024 + s * 128 + (ct - k * 128)
                v = buf[pl.ds(addr, _LANES)]
                gt = v[0]
                accg = accg + jnp.where(in_chunk, -_COEF_T * gt, 0.0)
                if h == 0:
                    v0 = buf[pl.ds(s * 128, _LANES)]
                    accg = accg + jnp.where(t != _PAD, _EPS * v0[0], 0.0)
            _sc_chunk_stats(buf, statm_v, stats_v, statp_v, h)
        base = (wid * _GROUPS + gi) * 256
        pltpu.sync_copy(statm_v, outm_hbm.at[pl.ds(base, 256)])
        pltpu.sync_copy(stats_v, outs_hbm.at[pl.ds(base, 256)])
        pltpu.sync_copy(statp_v, outp_hbm.at[pl.ds(base, 256)])
        return accg

    accg = lax.fori_loop(0, _GROUPS, group, jnp.float32(0.0))
    statm_v[pl.ds(0, _LANES)] = jnp.where(lane == 0, accg, 0.0)
    pltpu.sync_copy(statm_v.at[pl.ds(0, _LANES)],
                    outg_hbm.at[pl.ds(wid * _LANES, _LANES)])


@functools.lru_cache(maxsize=None)
def _sc_dense_fn():
    stats = jax.ShapeDtypeStruct((_NSC * 2 * _LANES,), jnp.float32)
    return pl.kernel(
        _sc_dense_body,
        mesh=plsc.VectorSubcoreMesh(core_axis_name="c", subcore_axis_name="s"),
        out_type=(stats, stats, stats,
                  jax.ShapeDtypeStruct((_NUM_WORKERS * _LANES,),
                                       jnp.float32)),
        scratch_types=[
            pltpu.VMEM((_CHUNK + _LANES,), jnp.float32),
            pltpu.VMEM((_GROUPS * _LANES,), jnp.int32),
            pltpu.VMEM((256,), jnp.float32),
            pltpu.VMEM((256,), jnp.float32),
            pltpu.VMEM((256,), jnp.float32),
            pltpu.VMEM((_LANES,), jnp.float32),
            pltpu.SemaphoreType.DMA,
            pltpu.SemaphoreType.DMA,
        ],
    )


def _tc_main_body(tgt_ref, x_ref, out_ref):
    i = pl.program_id(0)
    x = x_ref[...]
    m = jnp.max(x, axis=1, keepdims=True)
    se = jnp.sum(jnp.exp(x - m), axis=1, keepdims=True)
    lse = m + jnp.log(se)
    p = jnp.sum(x, axis=1, keepdims=True)
    tgt = tgt_ref[...]
    cols = lax.broadcasted_iota(jnp.int32, (_ROWS_PER_STEP, _V), 1)
    w = jnp.where(cols == tgt, -_COEF_T, 0.0) + jnp.where(cols == 0, _EPS, 0.0)
    g = jnp.sum(w * x, axis=1, keepdims=True)
    mask = (tgt != _PAD).astype(jnp.float32)
    part = jnp.sum(mask * (_C + lse - _EPS * p + g))
    prev = jnp.where(i == 0, 0.0, out_ref[0, 0])
    out_ref[0, 0] = prev + part


def _tc_main(pred, tgt2d):
    return pl.pallas_call(
        _tc_main_body,
        grid=(_NUM_STEPS,),
        in_specs=[
            pl.BlockSpec((_ROWS_PER_STEP, 1), lambda i: (i, 0)),
            pl.BlockSpec((_ROWS_PER_STEP, _V), lambda i: (i, 0)),
        ],
        out_specs=pl.BlockSpec(memory_space=pltpu.SMEM),
        out_shape=jax.ShapeDtypeStruct((1, 1), jnp.float32),
        compiler_params=pltpu.CompilerParams(
            dimension_semantics=("arbitrary",)),
    )(tgt2d, pred)


def _tc_combine_body(tc_ref, g_ref, m_ref, s_ref, p_ref, tgt_ref, out_ref):
    m2 = m_ref[...]
    mrow = jnp.max(m2, axis=1, keepdims=True)
    se = jnp.sum(s_ref[...] * jnp.exp(m2 - mrow), axis=1, keepdims=True)
    lse = mrow + jnp.log(se)
    p = jnp.sum(p_ref[...], axis=1, keepdims=True)
    mask = (tgt_ref[...] != _PAD).astype(jnp.float32)
    part = jnp.sum(mask * (_C + lse - _EPS * p))
    tot = tc_ref[0, 0] + jnp.sum(g_ref[...]) + part
    out_ref[0, 0] = tot * (1.0 / _N)


def _tc_combine(tc_scalar, sc_g, m2, s2, p2, tgt_sc):
    return pl.pallas_call(
        _tc_combine_body,
        grid=(1,),
        in_specs=[
            pl.BlockSpec(memory_space=pltpu.SMEM),
            pl.BlockSpec((_NUM_WORKERS, _LANES), lambda i: (0, 0)),
            pl.BlockSpec((_NSC, 2 * _LANES), lambda i: (0, 0)),
            pl.BlockSpec((_NSC, 2 * _LANES), lambda i: (0, 0)),
            pl.BlockSpec((_NSC, 2 * _LANES), lambda i: (0, 0)),
            pl.BlockSpec((_NSC, 1), lambda i: (_NTC // _NSC, 0)),
        ],
        out_specs=pl.BlockSpec(memory_space=pltpu.SMEM),
        out_shape=jax.ShapeDtypeStruct((1, 1), jnp.float32),
    )(tc_scalar, sc_g, m2, s2, p2, tgt_sc)


def _stats_2d(a):
    return a.reshape(_NSC, 2 * _LANES)


def kernel(pred, target):
    tgt2d = target.reshape(_N, 1)
    m2, s2, p2, sc_g = _sc_dense_fn()(pred, target)
    tc_scalar = _tc_main(pred, tgt2d)
    loss = _tc_combine(
        tc_scalar, sc_g.reshape(_NUM_WORKERS, _LANES),
        _stats_2d(m2), _stats_2d(s2), _stats_2d(p2), tgt2d)
    return loss[0, 0]


# speedup vs baseline: 1.1662x; 1.0499x over previous
import functools
import math

import jax
import jax.numpy as jnp
from jax import lax
from jax.experimental import pallas as pl
from jax.experimental.pallas import tpu as pltpu
from jax.experimental.pallas import tpu_sc as plsc

_N = 4096
_V = 32000
_PAD = 0
_SMOOTH = 0.1
_CONF = 1.0 - _SMOOTH
_EPS = _SMOOTH / (_V - 2)
_C = _CONF * math.log(_CONF) + _SMOOTH * math.log(_EPS)
_COEF_T = _CONF - _EPS

_LANES = 16
_NUM_WORKERS = 32

_NSC = 1024
_NTC = _N - _NSC

_ROWS_PER_STEP = 128
_NUM_STEPS = _NTC // _ROWS_PER_STEP

_ROWS_PER_WORKER = _NSC // _NUM_WORKERS
_GROUPS = _ROWS_PER_WORKER // 8
_CHUNKS_PER_WORKER = 2 * _GROUPS
_CHUNK = 4 * _V
_HALF = _V // 2
_TILES = _CHUNK // 1024


def _sc_chunk_stats(buf, statm_v, stats_v, statp_v, h):
    for sp in range(4):
        s0, s1 = 2 * sp, 2 * sp + 1

        def pass1(k, carry, s0=s0, s1=s1):
            m0, p0, m1, p1 = carry
            xs, ys = [], []
            for u in range(8):
                x = buf[pl.ds(k * 1024 + s0 * 128 + u * _LANES, _LANES)]
                y = buf[pl.ds(k * 1024 + s1 * 128 + u * _LANES, _LANES)]
                m0 = jnp.maximum(m0, x)
                m1 = jnp.maximum(m1, y)
                xs.append(x)
                ys.append(y)
            while len(xs) > 1:
                xs = [a + b for a, b in zip(xs[::2], xs[1::2])]
                ys = [a + b for a, b in zip(ys[::2], ys[1::2])]
            return m0, p0 + xs[0], m1, p1 + ys[0]

        zero = jnp.zeros((_LANES,), jnp.float32)
        ninf = jnp.full((_LANES,), -jnp.inf, jnp.float32)
        m0, p0, m1, p1 = lax.fori_loop(0, _TILES, pass1,
                                       (ninf, zero, ninf, zero))

        def pass2(k, carry, s0=s0, s1=s1, m0=m0, m1=m1):
            a0, a1 = carry
            xs, ys = [], []
            for u in range(8):
                x = buf[pl.ds(k * 1024 + s0 * 128 + u * _LANES, _LANES)]
                y = buf[pl.ds(k * 1024 + s1 * 128 + u * _LANES, _LANES)]
                xs.append(jnp.exp(x - m0))
                ys.append(jnp.exp(y - m1))
            while len(xs) > 1:
                xs = [a + b for a, b in zip(xs[::2], xs[1::2])]
                ys = [a + b for a, b in zip(ys[::2], ys[1::2])]
            return a0 + xs[0], a1 + ys[0]

        a0, a1 = lax.fori_loop(0, _TILES, pass2, (zero, zero))
        statm_v[pl.ds(s0 * 32 + h * _LANES, _LANES)] = m0
        stats_v[pl.ds(s0 * 32 + h * _LANES, _LANES)] = a0
        statp_v[pl.ds(s0 * 32 + h * _LANES, _LANES)] = p0
        statm_v[pl.ds(s1 * 32 + h * _LANES, _LANES)] = m1
        stats_v[pl.ds(s1 * 32 + h * _LANES, _LANES)] = a1
        statp_v[pl.ds(s1 * 32 + h * _LANES, _LANES)] = p1


def _bf16lanes(v, lane, op):
    for d in (1, 2, 4, 8):
        idx = jnp.bitwise_xor(lane, d)
        v = op(v, v.at[idx].get(mode="promise_in_bounds"))
    return v


def _sc_dense_body(pred_hbm, tgt_hbm, rest_hbm, se_hbm, outg_hbm,
                   buf, tgt_v, statm_v, stats_v, statp_v, rest_v, se_v,
                   sem, semt):
    wid = lax.axis_index("s") * 2 + lax.axis_index("c")
    row0 = _NTC + wid * _ROWS_PER_WORKER
    for gi in range(_GROUPS):
        pltpu.sync_copy(tgt_hbm.at[pl.ds(row0 + gi * 8, 8)],
                        tgt_v.at[pl.ds(gi * _LANES, 8)])
    lane = lax.iota(jnp.int32, _LANES)

    def group(gi, accg):
        tvec = tgt_v[pl.ds(gi * _LANES, _LANES)]
        for h in range(2):
            r0 = row0 + 8 * gi + 4 * h
            cps = [pltpu.async_copy(pred_hbm.at[r0 + rr],
                                    buf.at[pl.ds(rr * _V, _V)], sem)
                   for rr in range(4)]
            for cp in cps:
                cp.wait()
            for s in range(8):
                t = tvec[s]
                ct = t - _HALF * h
                in_chunk = (ct >= 0) & (ct < _HALF) & (t != _PAD)
                ct = jnp.where(in_chunk, ct, 0)
                k = ct // 128
                addr = k * 1024 + s * 128 + (ct - k * 128)
                v = buf[pl.ds(addr, _LANES)]
                gt = v[0]
                accg = accg + jnp.where(in_chunk, -_COEF_T * gt, 0.0)
                if h == 0:
                    v0 = buf[pl.ds(s * 128, _LANES)]
                    accg = accg + jnp.where(t != _PAD, _EPS * v0[0], 0.0)
            _sc_chunk_stats(buf, statm_v, stats_v, statp_v, h)
        rest_vec = jnp.zeros((_LANES,), jnp.float32)
        se_vec = jnp.where(lane < 8, 0.0, 1.0)
        for s in range(8):
            t = tvec[s]
            mask = t != _PAD
            m0 = statm_v[pl.ds(s * 32, _LANES)]
            m1 = statm_v[pl.ds(s * 32 + _LANES, _LANES)]
            a0 = stats_v[pl.ds(s * 32, _LANES)]
            a1 = stats_v[pl.ds(s * 32 + _LANES, _LANES)]
            p0 = statp_v[pl.ds(s * 32, _LANES)]
            p1 = statp_v[pl.ds(s * 32 + _LANES, _LANES)]
            mm = _bf16lanes(jnp.maximum(m0, m1), lane, jnp.maximum)
            se = _bf16lanes(a0 * jnp.exp(m0 - mm) + a1 * jnp.exp(m1 - mm),
                            lane, jnp.add)
            p = _bf16lanes(p0 + p1, lane, jnp.add)
            rest = jnp.where(mask, _C + mm - _EPS * p, 0.0)
            se = jnp.where(mask, se, 1.0)
            rest_vec = rest_vec + jnp.where(lane == s, rest, 0.0)
            se_vec = se_vec + jnp.where(lane == s, se, 0.0)
        rest_v[pl.ds(gi * _LANES, _LANES)] = rest_vec
        se_v[pl.ds(gi * _LANES, _LANES)] = se_vec
        return accg

    accg = lax.fori_loop(0, _GROUPS, group, jnp.float32(0.0))
    base = wid * _GROUPS * _LANES
    pltpu.sync_copy(rest_v, rest_hbm.at[pl.ds(base, _GROUPS * _LANES)])
    pltpu.sync_copy(se_v, se_hbm.at[pl.ds(base, _GROUPS * _LANES)])
    statm_v[pl.ds(0, _LANES)] = jnp.where(lane == 0, accg, 0.0)
    pltpu.sync_copy(statm_v.at[pl.ds(0, _LANES)],
                    outg_hbm.at[pl.ds(wid * _LANES, _LANES)])


@functools.lru_cache(maxsize=None)
def _sc_dense_fn():
    packed = jax.ShapeDtypeStruct((_NSC * 2,), jnp.float32)
    return pl.kernel(
        _sc_dense_body,
        mesh=plsc.VectorSubcoreMesh(core_axis_name="c", subcore_axis_name="s"),
        out_type=(packed, packed,
                  jax.ShapeDtypeStruct((_NUM_WORKERS * _LANES,),
                                       jnp.float32)),
        scratch_types=[
            pltpu.VMEM((_CHUNK + _LANES,), jnp.float32),
            pltpu.VMEM((_GROUPS * _LANES,), jnp.int32),
            pltpu.VMEM((256,), jnp.float32),
            pltpu.VMEM((256,), jnp.float32),
            pltpu.VMEM((256,), jnp.float32),
            pltpu.VMEM((_GROUPS * _LANES,), jnp.float32),
            pltpu.VMEM((_GROUPS * _LANES,), jnp.float32),
            pltpu.SemaphoreType.DMA,
            pltpu.SemaphoreType.DMA,
        ],
    )


def _tc_main_body(tgt_ref, x_ref, out_ref):
    i = pl.program_id(0)
    x = x_ref[...]
    m = jnp.max(x, axis=1, keepdims=True)
    se = jnp.sum(jnp.exp(x - m), axis=1, keepdims=True)
    lse = m + jnp.log(se)
    p = jnp.sum(x, axis=1, keepdims=True)
    tgt = tgt_ref[...]
    cols = lax.broadcasted_iota(jnp.int32, (_ROWS_PER_STEP, _V), 1)
    w = jnp.where(cols == tgt, -_COEF_T, 0.0) + jnp.where(cols == 0, _EPS, 0.0)
    g = jnp.sum(w * x, axis=1, keepdims=True)
    mask = (tgt != _PAD).astype(jnp.float32)
    part = jnp.sum(mask * (_C + lse - _EPS * p + g))
    prev = jnp.where(i == 0, 0.0, out_ref[0, 0])
    out_ref[0, 0] = prev + part


def _tc_main(pred, tgt2d):
    return pl.pallas_call(
        _tc_main_body,
        grid=(_NUM_STEPS,),
        in_specs=[
            pl.BlockSpec((_ROWS_PER_STEP, 1), lambda i: (i, 0)),
            pl.BlockSpec((_ROWS_PER_STEP, _V), lambda i: (i, 0)),
        ],
        out_specs=pl.BlockSpec(memory_space=pltpu.SMEM),
        out_shape=jax.ShapeDtypeStruct((1, 1), jnp.float32),
        compiler_params=pltpu.CompilerParams(
            dimension_semantics=("arbitrary",)),
    )(tgt2d, pred)


def _tc_combine_body(tc_ref, g_ref, rest_ref, se_ref, out_ref):
    part = jnp.sum(rest_ref[...]) + jnp.sum(jnp.log(se_ref[...]))
    tot = tc_ref[0, 0] + jnp.sum(g_ref[...]) + part
    out_ref[0, 0] = tot * (1.0 / _N)


def _tc_combine(tc_scalar, sc_g, rest2, se2):
    return pl.pallas_call(
        _tc_combine_body,
        grid=(1,),
        in_specs=[
            pl.BlockSpec(memory_space=pltpu.SMEM),
            pl.BlockSpec((4, 128), lambda i: (0, 0)),
            pl.BlockSpec((_NSC * 2 // 128, 128), lambda i: (0, 0)),
            pl.BlockSpec((_NSC * 2 // 128, 128), lambda i: (0, 0)),
        ],
        out_specs=pl.BlockSpec(memory_space=pltpu.SMEM),
        out_shape=jax.ShapeDtypeStruct((1, 1), jnp.float32),
    )(tc_scalar, sc_g, rest2, se2)


def kernel(pred, target):
    tgt2d = target.reshape(_N, 1)
    rest, se, sc_g = _sc_dense_fn()(pred, target)
    tc_scalar = _tc_main(pred, tgt2d)
    loss = _tc_combine(tc_scalar, sc_g.reshape(4, 128),
                       rest.reshape(_NSC * 2 // 128, 128),
                       se.reshape(_NSC * 2 // 128, 128))
    return loss[0, 0]


# speedup vs baseline: 1.1972x; 1.0266x over previous
import functools
import math

import jax
import jax.numpy as jnp
from jax import lax
from jax.experimental import pallas as pl
from jax.experimental.pallas import tpu as pltpu
from jax.experimental.pallas import tpu_sc as plsc

_N = 4096
_V = 32000
_PAD = 0
_SMOOTH = 0.1
_CONF = 1.0 - _SMOOTH
_EPS = _SMOOTH / (_V - 2)
_C = _CONF * math.log(_CONF) + _SMOOTH * math.log(_EPS)
_COEF_T = _CONF - _EPS

_LANES = 16
_NUM_WORKERS = 32

_NSC = 1024
_NTC = _N - _NSC

_ROWS_PER_STEP = 192
_NUM_STEPS = _NTC // _ROWS_PER_STEP

_ROWS_PER_WORKER = _NSC // _NUM_WORKERS
_GROUPS = _ROWS_PER_WORKER // 8
_CHUNKS_PER_WORKER = 2 * _GROUPS
_CHUNK = 4 * _V
_HALF = _V // 2
_TILES = _CHUNK // 1024


def _sc_chunk_stats(buf, statm_v, stats_v, statp_v, h):
    for sp in range(4):
        s0, s1 = 2 * sp, 2 * sp + 1

        def pass1(k, carry, s0=s0, s1=s1):
            m0, p0, m1, p1 = carry
            xs, ys = [], []
            for u in range(8):
                x = buf[pl.ds(k * 1024 + s0 * 128 + u * _LANES, _LANES)]
                y = buf[pl.ds(k * 1024 + s1 * 128 + u * _LANES, _LANES)]
                m0 = jnp.maximum(m0, x)
                m1 = jnp.maximum(m1, y)
                xs.append(x)
                ys.append(y)
            while len(xs) > 1:
                xs = [a + b for a, b in zip(xs[::2], xs[1::2])]
                ys = [a + b for a, b in zip(ys[::2], ys[1::2])]
            return m0, p0 + xs[0], m1, p1 + ys[0]

        zero = jnp.zeros((_LANES,), jnp.float32)
        ninf = jnp.full((_LANES,), -jnp.inf, jnp.float32)
        m0, p0, m1, p1 = lax.fori_loop(0, _TILES, pass1,
                                       (ninf, zero, ninf, zero))

        def pass2(k, carry, s0=s0, s1=s1, m0=m0, m1=m1):
            a0, a1 = carry
            xs, ys = [], []
            for u in range(8):
                x = buf[pl.ds(k * 1024 + s0 * 128 + u * _LANES, _LANES)]
                y = buf[pl.ds(k * 1024 + s1 * 128 + u * _LANES, _LANES)]
                xs.append(jnp.exp(x - m0))
                ys.append(jnp.exp(y - m1))
            while len(xs) > 1:
                xs = [a + b for a, b in zip(xs[::2], xs[1::2])]
                ys = [a + b for a, b in zip(ys[::2], ys[1::2])]
            return a0 + xs[0], a1 + ys[0]

        a0, a1 = lax.fori_loop(0, _TILES, pass2, (zero, zero))
        statm_v[pl.ds(s0 * 32 + h * _LANES, _LANES)] = m0
        stats_v[pl.ds(s0 * 32 + h * _LANES, _LANES)] = a0
        statp_v[pl.ds(s0 * 32 + h * _LANES, _LANES)] = p0
        statm_v[pl.ds(s1 * 32 + h * _LANES, _LANES)] = m1
        stats_v[pl.ds(s1 * 32 + h * _LANES, _LANES)] = a1
        statp_v[pl.ds(s1 * 32 + h * _LANES, _LANES)] = p1


def _bf16lanes(v, lane, op):
    for d in (1, 2, 4, 8):
        idx = jnp.bitwise_xor(lane, d)
        v = op(v, v.at[idx].get(mode="promise_in_bounds"))
    return v


def _sc_dense_body(pred_hbm, tgt_hbm, rest_hbm, se_hbm, outg_hbm,
                   buf, tgt_v, statm_v, stats_v, statp_v, rest_v, se_v,
                   sem, semt):
    wid = lax.axis_index("s") * 2 + lax.axis_index("c")
    row0 = _NTC + wid * _ROWS_PER_WORKER
    for gi in range(_GROUPS):
        pltpu.sync_copy(tgt_hbm.at[pl.ds(row0 + gi * 8, 8)],
                        tgt_v.at[pl.ds(gi * _LANES, 8)])
    lane = lax.iota(jnp.int32, _LANES)

    def group(gi, accg):
        tvec = tgt_v[pl.ds(gi * _LANES, _LANES)]
        for h in range(2):
            r0 = row0 + 8 * gi + 4 * h
            cps = [pltpu.async_copy(pred_hbm.at[r0 + rr],
                                    buf.at[pl.ds(rr * _V, _V)], sem)
                   for rr in range(4)]
            for cp in cps:
                cp.wait()
            for s in range(8):
                t = tvec[s]
                ct = t - _HALF * h
                in_chunk = (ct >= 0) & (ct < _HALF) & (t != _PAD)
                ct = jnp.where(in_chunk, ct, 0)
                k = ct // 128
                addr = k * 1024 + s * 128 + (ct - k * 128)
                v = buf[pl.ds(addr, _LANES)]
                gt = v[0]
                accg = accg + jnp.where(in_chunk, -_COEF_T * gt, 0.0)
                if h == 0:
                    v0 = buf[pl.ds(s * 128, _LANES)]
                    accg = accg + jnp.where(t != _PAD, _EPS * v0[0], 0.0)
            _sc_chunk_stats(buf, statm_v, stats_v, statp_v, h)
        rest_vec = jnp.zeros((_LANES,), jnp.float32)
        se_vec = jnp.where(lane < 8, 0.0, 1.0)
        for s in range(8):
            t = tvec[s]
            mask = t != _PAD
            m0 = statm_v[pl.ds(s * 32, _LANES)]
            m1 = statm_v[pl.ds(s * 32 + _LANES, _LANES)]
            a0 = stats_v[pl.ds(s * 32, _LANES)]
            a1 = stats_v[pl.ds(s * 32 + _LANES, _LANES)]
            p0 = statp_v[pl.ds(s * 32, _LANES)]
            p1 = statp_v[pl.ds(s * 32 + _LANES, _LANES)]
            mm = _bf16lanes(jnp.maximum(m0, m1), lane, jnp.maximum)
            se = _bf16lanes(a0 * jnp.exp(m0 - mm) + a1 * jnp.exp(m1 - mm),
                            lane, jnp.add)
            p = _bf16lanes(p0 + p1, lane, jnp.add)
            rest = jnp.where(mask, _C + mm - _EPS * p, 0.0)
            se = jnp.where(mask, se, 1.0)
            rest_vec = rest_vec + jnp.where(lane == s, rest, 0.0)
            se_vec = se_vec + jnp.where(lane == s, se, 0.0)
        rest_v[pl.ds(gi * _LANES, _LANES)] = rest_vec
        se_v[pl.ds(gi * _LANES, _LANES)] = se_vec
        return accg

    accg = lax.fori_loop(0, _GROUPS, group, jnp.float32(0.0))
    base = wid * _GROUPS * _LANES
    pltpu.sync_copy(rest_v, rest_hbm.at[pl.ds(base, _GROUPS * _LANES)])
    pltpu.sync_copy(se_v, se_hbm.at[pl.ds(base, _GROUPS * _LANES)])
    statm_v[pl.ds(0, _LANES)] = jnp.where(lane == 0, accg, 0.0)
    pltpu.sync_copy(statm_v.at[pl.ds(0, _LANES)],
                    outg_hbm.at[pl.ds(wid * _LANES, _LANES)])


@functools.lru_cache(maxsize=None)
def _sc_dense_fn():
    packed = jax.ShapeDtypeStruct((_NSC * 2,), jnp.float32)
    return pl.kernel(
        _sc_dense_body,
        mesh=plsc.VectorSubcoreMesh(core_axis_name="c", subcore_axis_name="s"),
        out_type=(packed, packed,
                  jax.ShapeDtypeStruct((_NUM_WORKERS * _LANES,),
                                       jnp.float32)),
        scratch_types=[
            pltpu.VMEM((_CHUNK + _LANES,), jnp.float32),
            pltpu.VMEM((_GROUPS * _LANES,), jnp.int32),
            pltpu.VMEM((256,), jnp.float32),
            pltpu.VMEM((256,), jnp.float32),
            pltpu.VMEM((256,), jnp.float32),
            pltpu.VMEM((_GROUPS * _LANES,), jnp.float32),
            pltpu.VMEM((_GROUPS * _LANES,), jnp.float32),
            pltpu.SemaphoreType.DMA,
            pltpu.SemaphoreType.DMA,
        ],
    )


def _tc_main_body(tgt_ref, x_ref, out_ref):
    i = pl.program_id(0)
    x = x_ref[...]
    m = jnp.max(x, axis=1, keepdims=True)
    se = jnp.sum(jnp.exp(x - m), axis=1, keepdims=True)
    lse = m + jnp.log(se)
    p = jnp.sum(x, axis=1, keepdims=True)
    tgt = tgt_ref[...]
    cols = lax.broadcasted_iota(jnp.int32, (_ROWS_PER_STEP, _V), 1)
    w = jnp.where(cols == tgt, -_COEF_T, 0.0) + jnp.where(cols == 0, _EPS, 0.0)
    g = jnp.sum(w * x, axis=1, keepdims=True)
    mask = (tgt != _PAD).astype(jnp.float32)
    part = jnp.sum(mask * (_C + lse - _EPS * p + g))
    prev = jnp.where(i == 0, 0.0, out_ref[0, 0])
    out_ref[0, 0] = prev + part


def _tc_main(pred, tgt2d):
    return pl.pallas_call(
        _tc_main_body,
        grid=(_NUM_STEPS,),
        in_specs=[
            pl.BlockSpec((_ROWS_PER_STEP, 1), lambda i: (i, 0)),
            pl.BlockSpec((_ROWS_PER_STEP, _V), lambda i: (i, 0)),
        ],
        out_specs=pl.BlockSpec(memory_space=pltpu.SMEM),
        out_shape=jax.ShapeDtypeStruct((1, 1), jnp.float32),
        compiler_params=pltpu.CompilerParams(
            dimension_semantics=("arbitrary",)),
    )(tgt2d, pred)


def _tc_combine_body(tc_ref, g_ref, rest_ref, se_ref, out_ref):
    part = jnp.sum(rest_ref[...]) + jnp.sum(jnp.log(se_ref[...]))
    tot = tc_ref[0, 0] + jnp.sum(g_ref[...]) + part
    out_ref[0, 0] = tot * (1.0 / _N)


def _tc_combine(tc_scalar, sc_g, rest2, se2):
    return pl.pallas_call(
        _tc_combine_body,
        grid=(1,),
        in_specs=[
            pl.BlockSpec(memory_space=pltpu.SMEM),
            pl.BlockSpec((4, 128), lambda i: (0, 0)),
            pl.BlockSpec((_NSC * 2 // 128, 128), lambda i: (0, 0)),
            pl.BlockSpec((_NSC * 2 // 128, 128), lambda i: (0, 0)),
        ],
        out_specs=pl.BlockSpec(memory_space=pltpu.SMEM),
        out_shape=jax.ShapeDtypeStruct((1, 1), jnp.float32),
    )(tc_scalar, sc_g, rest2, se2)


def kernel(pred, target):
    tgt2d = target.reshape(_N, 1)
    rest, se, sc_g = _sc_dense_fn()(pred, target)
    tc_scalar = _tc_main(pred, tgt2d)
    loss = _tc_combine(tc_scalar, sc_g.reshape(4, 128),
                       rest.reshape(_NSC * 2 // 128, 128),
                       se.reshape(_NSC * 2 // 128, 128))
    return loss[0, 0]
